# Initial kernel scaffold; baseline (speedup 1.0000x reference)
import jax
import jax.numpy as jnp
from jax.experimental import pallas as pl


def kernel(num_prop, cat_prop, tweet_range_list, edge_index, re_index, Wn, bn, Wc, bc, Wt, bt, att_l, att_r, Wf1, bf1, Wl, bl):
    raise NotImplementedError("write your pallas kernel here")


# speedup vs baseline: 1.9652x; 1.9652x over previous
import functools

import jax
import jax.numpy as jnp
from jax import lax
from jax.experimental import pallas as pl
from jax.experimental.pallas import tpu as pltpu
from jax.experimental.pallas import tpu_sc as plsc

N = 100000
E = 1600000
B = 5000
NUM_USERS = 10000
D1 = 32
H = D1 // 2
EPS = 0.1

_RB = 2000
_GRID = N // _RB

_f32 = jnp.float32


def _leaky(x):
    return jnp.where(x > 0, x, 0.01 * x)


def _rowspec(width):
    return pl.BlockSpec((_RB, width), lambda i: (i, 0))


def _fullspec(a, b):
    return pl.BlockSpec((a, b), lambda i: (0, 0))


def _feat_body(num_ref, cat_ref, wn_ref, bn_ref, wc_ref, bc_ref, wt_ref,
               bt_ref, attl_ref, attr_ref, xlo_ref, xhi_ref, al_ref, ar_ref):
    h1 = _leaky(jnp.dot(num_ref[...], wn_ref[...],
                        preferred_element_type=_f32) + bn_ref[...])
    h2 = _leaky(jnp.dot(cat_ref[...], wc_ref[...],
                        preferred_element_type=_f32) + bc_ref[...])
    x = jnp.concatenate([h1, h2], axis=1)
    x = _leaky(jnp.dot(x, wt_ref[...], preferred_element_type=_f32)
               + bt_ref[...])
    xlo_ref[...] = x[:, :H]
    xhi_ref[...] = x[:, H:]
    al_ref[...] = jnp.sum(x * attl_ref[...], axis=1, keepdims=True)
    ar_ref[...] = jnp.sum(x * attr_ref[...], axis=1, keepdims=True)


def _tc_feat(num, cat, Wn, bn, Wc, bc, Wt, bt, attl_row, attr_row):
    return pl.pallas_call(
        _feat_body,
        grid=(_GRID,),
        in_specs=[_rowspec(16), _rowspec(16), _fullspec(16, 16),
                  _fullspec(1, 16), _fullspec(16, 16), _fullspec(1, 16),
                  _fullspec(32, 32), _fullspec(1, 32), _fullspec(1, 32),
                  _fullspec(1, 32)],
        out_specs=[_rowspec(H), _rowspec(H), _rowspec(1), _rowspec(1)],
        out_shape=[jax.ShapeDtypeStruct((N, H), _f32),
                   jax.ShapeDtypeStruct((N, H), _f32),
                   jax.ShapeDtypeStruct((N, 1), _f32),
                   jax.ShapeDtypeStruct((N, 1), _f32)],
    )(num, cat, Wn, bn, Wc, bc, Wt, bt, attl_row, attr_row)


def _prep_body(h0_ref, h1_ref, al_ref, ar_ref,
               srct_ref, dstt_ref, dis_ref, sl_ref):
    deg = h0_ref[...] + h1_ref[...] + 1.0
    dis = lax.rsqrt(deg)
    al = al_ref[...]
    ar = ar_ref[...]
    srct_ref[...] = jnp.concatenate([al, dis], axis=1)
    dstt_ref[...] = jnp.concatenate([ar, dis], axis=1)
    dis_ref[...] = dis
    sl_ref[...] = jnp.tanh(al + ar) / deg


def _tc_prep(h0, h1, al, ar):
    return pl.pallas_call(
        _prep_body,
        grid=(_GRID,),
        in_specs=[_rowspec(1)] * 4,
        out_specs=[_rowspec(2), _rowspec(2), _rowspec(1), _rowspec(1)],
        out_shape=[jax.ShapeDtypeStruct((N, 2), _f32),
                   jax.ShapeDtypeStruct((N, 2), _f32),
                   jax.ShapeDtypeStruct((N, 1), _f32),
                   jax.ShapeDtypeStruct((N, 1), _f32)],
    )(h0, h1, al, ar)


def _mix_body(aglo_ref, aghi_ref, xlo_ref, xhi_ref, sl_ref, dis_ref,
              attl_ref, attr_ref,
              ylo_ref, yhi_ref, srct_ref, dstt_ref, sl2_ref):
    s = sl_ref[...] + EPS
    ylo = aglo_ref[...] + xlo_ref[...] * s
    yhi = aghi_ref[...] + xhi_ref[...] * s
    ylo_ref[...] = ylo
    yhi_ref[...] = yhi
    y = jnp.concatenate([ylo, yhi], axis=1)
    al2 = jnp.sum(y * attl_ref[...], axis=1, keepdims=True)
    ar2 = jnp.sum(y * attr_ref[...], axis=1, keepdims=True)
    dis = dis_ref[...]
    srct_ref[...] = jnp.concatenate([al2, dis], axis=1)
    dstt_ref[...] = jnp.concatenate([ar2, dis], axis=1)
    sl2_ref[...] = jnp.tanh(al2 + ar2) * dis * dis


def _tc_mix(aglo, aghi, xlo, xhi, sl, dis, attl_row, attr_row):
    return pl.pallas_call(
        _mix_body,
        grid=(_GRID,),
        in_specs=[_rowspec(H), _rowspec(H), _rowspec(H), _rowspec(H),
                  _rowspec(1), _rowspec(1), _fullspec(1, 32), _fullspec(1, 32)],
        out_specs=[_rowspec(H), _rowspec(H), _rowspec(2), _rowspec(2),
                   _rowspec(1)],
        out_shape=[jax.ShapeDtypeStruct((N, H), _f32),
                   jax.ShapeDtypeStruct((N, H), _f32),
                   jax.ShapeDtypeStruct((N, 2), _f32),
                   jax.ShapeDtypeStruct((N, 2), _f32),
                   jax.ShapeDtypeStruct((N, 1), _f32)],
    )(aglo, aghi, xlo, xhi, sl, dis, attl_row, attr_row)


def _x2_body(aglo_ref, aghi_ref, x1lo_ref, x1hi_ref, xlo_ref, xhi_ref,
             sl2_ref, x2_ref):
    s = sl2_ref[...]
    lo = aglo_ref[...] + x1lo_ref[...] * s + EPS * xlo_ref[...]
    hi = aghi_ref[...] + x1hi_ref[...] * s + EPS * xhi_ref[...]
    x2 = jnp.concatenate([lo, hi], axis=1)
    x2_ref[...] = jnp.sqrt(x2 * x2 + 1e-8)


def _tc_x2(aglo, aghi, x1lo, x1hi, xlo, xhi, sl2):
    return pl.pallas_call(
        _x2_body,
        grid=(_GRID,),
        in_specs=[_rowspec(H)] * 4 + [_rowspec(H), _rowspec(H), _rowspec(1)],
        out_specs=[pl.BlockSpec((_RB, D1), lambda i: (i, 0))],
        out_shape=[jax.ShapeDtypeStruct((N, D1), _f32)],
    )(aglo, aghi, x1lo, x1hi, xlo, xhi, sl2)[0]


def _mlp_body(x_ref, wf_ref, bf_ref, wl_ref, bl_ref, o_ref):
    h = _leaky(jnp.dot(x_ref[...], wf_ref[...],
                       preferred_element_type=_f32) + bf_ref[...])
    o_ref[...] = jnp.dot(h, wl_ref[...], preferred_element_type=_f32) \
        + bl_ref[...]


def _tc_mlp(x3g, Wf1, bf1_row, Wl, bl_row):
    return pl.pallas_call(
        _mlp_body,
        grid=(1,),
        in_specs=[pl.BlockSpec((NUM_USERS, D1), lambda i: (0, 0)),
                  _fullspec(D1, H), _fullspec(1, H), _fullspec(H, 2),
                  _fullspec(1, 2)],
        out_specs=[pl.BlockSpec((NUM_USERS, 2), lambda i: (0, 0))],
        out_shape=[jax.ShapeDtypeStruct((NUM_USERS, 2), _f32)],
    )(x3g, Wf1, bf1_row, Wl, bl_row)[0]


def _sc_hist(dst):
    h = jnp.zeros((N,), _f32).at[dst].add(1.0)
    return h.reshape(N, 1), jnp.zeros((N, 1), _f32)


def _sc_w(src, dst, srct, dstt):
    a = srct[src]
    b = dstt[dst]
    return jnp.tanh(a[:, 0] + b[:, 0]) * (a[:, 1] * b[:, 1])


def _sc_edge(src, dst, w, xlo, xhi):
    mlo = xlo[src] * w[:, None]
    mhi = xhi[src] * w[:, None]
    aglo = jnp.zeros((N, H), _f32).at[dst].add(mlo)
    aghi = jnp.zeros((N, H), _f32).at[dst].add(mhi)
    return aglo, aghi


def _sc_seg(x2, tr):
    seg = jnp.clip(jnp.searchsorted(tr, jnp.arange(N), side='right') - 1,
                   0, B - 1)
    return jax.ops.segment_sum(x2, seg, num_segments=B)


def _sc_regather(x3p, re_index):
    return x3p[re_index]


def kernel(num_prop, cat_prop, tweet_range_list, edge_index, re_index,
           Wn, bn, Wc, bc, Wt, bt, att_l, att_r, Wf1, bf1, Wl, bl):
    attl_row = att_l.reshape(1, D1)
    attr_row = att_r.reshape(1, D1)
    src = edge_index[0]
    dst = edge_index[1]

    xlo, xhi, al, ar = _tc_feat(num_prop, cat_prop, Wn, bn.reshape(1, 16),
                                Wc, bc.reshape(1, 16), Wt, bt.reshape(1, 32),
                                attl_row, attr_row)
    h0, h1 = _sc_hist(dst)
    srct, dstt, dis, sl1 = _tc_prep(h0, h1, al, ar)

    w1 = _sc_w(src, dst, srct, dstt)
    aglo, aghi = _sc_edge(src, dst, w1, xlo, xhi)
    x1lo, x1hi, srct2, dstt2, sl2 = _tc_mix(aglo, aghi, xlo, xhi, sl1, dis,
                                            attl_row, attr_row)

    w2 = _sc_w(src, dst, srct2, dstt2)
    ag2lo, ag2hi = _sc_edge(src, dst, w2, x1lo, x1hi)
    x2 = _tc_x2(ag2lo, ag2hi, x1lo, x1hi, xlo, xhi, sl2)

    x3 = _sc_seg(x2, tweet_range_list)
    x3p = jnp.concatenate([x3, jnp.zeros((NUM_USERS - B, D1), _f32)], axis=0)
    x3g = _sc_regather(x3p, re_index)
    return _tc_mlp(x3g, Wf1, bf1.reshape(1, H), Wl, bl.reshape(1, 2))


# speedup vs baseline: 8.8592x; 4.5081x over previous
import functools

import jax
import jax.numpy as jnp
import numpy as np
from jax import lax
from jax.experimental import pallas as pl
from jax.experimental.pallas import tpu as pltpu
from jax.experimental.pallas import tpu_sc as plsc

N = 100000
E = 1600000
B = 5000
NUM_USERS = 10000
D1 = 32
H = D1 // 2
EPS = 0.1

_RB = 2000
_GRID = N // _RB

_f32 = jnp.float32


def _leaky(x):
    return jnp.where(x > 0, x, 0.01 * x)


def _rowspec(width):
    return pl.BlockSpec((_RB, width), lambda i: (i, 0))


def _fullspec(a, b):
    return pl.BlockSpec((a, b), lambda i: (0, 0))


def _feat_body(num_ref, cat_ref, wn_ref, bn_ref, wc_ref, bc_ref, wt_ref,
               bt_ref, attl_ref, attr_ref, xlo_ref, xhi_ref, al_ref, ar_ref):
    h1 = _leaky(jnp.dot(num_ref[...], wn_ref[...],
                        preferred_element_type=_f32) + bn_ref[...])
    h2 = _leaky(jnp.dot(cat_ref[...], wc_ref[...],
                        preferred_element_type=_f32) + bc_ref[...])
    x = jnp.concatenate([h1, h2], axis=1)
    x = _leaky(jnp.dot(x, wt_ref[...], preferred_element_type=_f32)
               + bt_ref[...])
    xlo_ref[...] = x[:, :H]
    xhi_ref[...] = x[:, H:]
    al_ref[...] = jnp.sum(x * attl_ref[...], axis=1, keepdims=True)
    ar_ref[...] = jnp.sum(x * attr_ref[...], axis=1, keepdims=True)


def _tc_feat(num, cat, Wn, bn, Wc, bc, Wt, bt, attl_row, attr_row):
    return pl.pallas_call(
        _feat_body,
        grid=(_GRID,),
        in_specs=[_rowspec(16), _rowspec(16), _fullspec(16, 16),
                  _fullspec(1, 16), _fullspec(16, 16), _fullspec(1, 16),
                  _fullspec(32, 32), _fullspec(1, 32), _fullspec(1, 32),
                  _fullspec(1, 32)],
        out_specs=[_rowspec(H), _rowspec(H), _rowspec(1), _rowspec(1)],
        out_shape=[jax.ShapeDtypeStruct((N, H), _f32),
                   jax.ShapeDtypeStruct((N, H), _f32),
                   jax.ShapeDtypeStruct((N, 1), _f32),
                   jax.ShapeDtypeStruct((N, 1), _f32)],
    )(num, cat, Wn, bn, Wc, bc, Wt, bt, attl_row, attr_row)


def _prep_body(h0_ref, h1_ref, al_ref, ar_ref, dis_ref, sl_ref):
    deg = h0_ref[...] + h1_ref[...] + 1.0
    dis = lax.rsqrt(deg)
    dis_ref[...] = dis
    sl_ref[...] = jnp.tanh(al_ref[...] + ar_ref[...]) / deg


def _tc_prep(h0, h1, al, ar):
    return pl.pallas_call(
        _prep_body,
        grid=(_GRID,),
        in_specs=[_rowspec(1)] * 4,
        out_specs=[_rowspec(1), _rowspec(1)],
        out_shape=[jax.ShapeDtypeStruct((N, 1), _f32),
                   jax.ShapeDtypeStruct((N, 1), _f32)],
    )(h0, h1, al, ar)


def _mix_body(aglo_ref, aghi_ref, xlo_ref, xhi_ref, sl_ref, dis_ref,
              attl_ref, attr_ref,
              ylo_ref, yhi_ref, al2_ref, ar2_ref, sl2_ref):
    s = sl_ref[...] + EPS
    ylo = aglo_ref[...] + xlo_ref[...] * s
    yhi = aghi_ref[...] + xhi_ref[...] * s
    ylo_ref[...] = ylo
    yhi_ref[...] = yhi
    y = jnp.concatenate([ylo, yhi], axis=1)
    al2 = jnp.sum(y * attl_ref[...], axis=1, keepdims=True)
    ar2 = jnp.sum(y * attr_ref[...], axis=1, keepdims=True)
    dis = dis_ref[...]
    al2_ref[...] = al2
    ar2_ref[...] = ar2
    sl2_ref[...] = jnp.tanh(al2 + ar2) * dis * dis


def _tc_mix(aglo, aghi, xlo, xhi, sl, dis, attl_row, attr_row):
    return pl.pallas_call(
        _mix_body,
        grid=(_GRID,),
        in_specs=[_rowspec(H), _rowspec(H), _rowspec(H), _rowspec(H),
                  _rowspec(1), _rowspec(1), _fullspec(1, 32), _fullspec(1, 32)],
        out_specs=[_rowspec(H), _rowspec(H), _rowspec(1), _rowspec(1),
                   _rowspec(1)],
        out_shape=[jax.ShapeDtypeStruct((N, H), _f32),
                   jax.ShapeDtypeStruct((N, H), _f32),
                   jax.ShapeDtypeStruct((N, 1), _f32),
                   jax.ShapeDtypeStruct((N, 1), _f32),
                   jax.ShapeDtypeStruct((N, 1), _f32)],
    )(aglo, aghi, xlo, xhi, sl, dis, attl_row, attr_row)


def _x2_body(aglo_ref, aghi_ref, x1lo_ref, x1hi_ref, xlo_ref, xhi_ref,
             sl2_ref, x2_ref):
    s = sl2_ref[...]
    lo = aglo_ref[...] + x1lo_ref[...] * s + EPS * xlo_ref[...]
    hi = aghi_ref[...] + x1hi_ref[...] * s + EPS * xhi_ref[...]
    x2 = jnp.concatenate([lo, hi], axis=1)
    x2_ref[...] = jnp.sqrt(x2 * x2 + 1e-8)


def _tc_x2(aglo, aghi, x1lo, x1hi, xlo, xhi, sl2):
    return pl.pallas_call(
        _x2_body,
        grid=(_GRID,),
        in_specs=[_rowspec(H)] * 4 + [_rowspec(H), _rowspec(H), _rowspec(1)],
        out_specs=[pl.BlockSpec((_RB, D1), lambda i: (i, 0))],
        out_shape=[jax.ShapeDtypeStruct((N, D1), _f32)],
    )(aglo, aghi, x1lo, x1hi, xlo, xhi, sl2)[0]


def _mlp_body(x_ref, wf_ref, bf_ref, wl_ref, bl_ref, o_ref):
    h = _leaky(jnp.dot(x_ref[...], wf_ref[...],
                       preferred_element_type=_f32) + bf_ref[...])
    o_ref[...] = jnp.dot(h, wl_ref[...], preferred_element_type=_f32) \
        + bl_ref[...]


def _tc_mlp(x3g, Wf1, bf1_row, Wl, bl_row):
    return pl.pallas_call(
        _mlp_body,
        grid=(1,),
        in_specs=[pl.BlockSpec((NUM_USERS, D1), lambda i: (0, 0)),
                  _fullspec(D1, H), _fullspec(1, H), _fullspec(H, 2),
                  _fullspec(1, 2)],
        out_specs=[pl.BlockSpec((NUM_USERS, 2), lambda i: (0, 0))],
        out_shape=[jax.ShapeDtypeStruct((NUM_USERS, 2), _f32)],
    )(x3g, Wf1, bf1_row, Wl, bl_row)[0]


_NC, _NS, _L = 2, 16, 16
_CH = 128
_NCHUNK = E // _CH
_NPT = N // _NS
_MESH = dict(mesh=plsc.VectorSubcoreMesh(core_axis_name="c",
                                         subcore_axis_name="s"))
_NOTILE = dict(compiler_params=pltpu.CompilerParams(
    use_tc_tiling_on_sc=False))
_NOLAYOUT = dict(compiler_params=pltpu.CompilerParams(
    needs_layout_passes=False))


def _zero_rows(buf, nrows, width16):
    z = jnp.zeros((16,), _f32)

    def body(j, _):
        for col in range(width16):
            buf[j, pl.ds(col * 16, 16)] = z
        return 0

    lax.fori_loop(0, nrows, body, 0)


def _hist_body(dst_hbm, h0_hbm, h1_hbm, dst_v, ones_v, zb_v, acc, sem):
    c = lax.axis_index("c")
    s = lax.axis_index("s")
    def fill16(j, _):
        zb_v[pl.ds(j * 16, 16)] = jnp.zeros((16,), _f32)
        return 0
    lax.fori_loop(0, 391, fill16, 0)
    for j in range(8):
        ones_v[pl.ds(j * 16, 16)] = jnp.ones((16,), _f32)
    start_a = jnp.minimum((s * _NPT) // 8 * 8, N - 6256)
    pltpu.sync_copy(zb_v, acc.at[pl.ds(start_a, 6256)])
    plsc.subcore_barrier()

    half = _NCHUNK // 2
    nch = 390 + jnp.where(s < half - 16 * 390, 1, 0)

    def chunk(i, _):
        base = (c * half + s + i * _NS) * _CH
        pltpu.sync_copy(dst_hbm.at[pl.ds(base, _CH)], dst_v)
        pltpu.sync_copy(ones_v, acc.at[dst_v], add=True)
        return 0

    lax.fori_loop(0, nch, chunk, 0)
    plsc.subcore_barrier()

    pltpu.sync_copy(acc.at[pl.ds(start_a, 6256)], zb_v)

    @pl.when(c == 0)
    def _():
        pltpu.sync_copy(zb_v, h0_hbm.at[pl.ds(start_a, 6256)])

    @pl.when(c == 1)
    def _():
        pltpu.sync_copy(zb_v, h1_hbm.at[pl.ds(start_a, 6256)])


_hist_call = functools.partial(
    pl.kernel, _hist_body,
    out_type=[jax.ShapeDtypeStruct((N,), _f32),
              jax.ShapeDtypeStruct((N,), _f32)],
    scratch_types=[pltpu.VMEM((_CH,), jnp.int32),
                   pltpu.VMEM((_CH,), _f32),
                   pltpu.VMEM((6256,), _f32),
                   pltpu.VMEM_SHARED((N,), _f32),
                   pltpu.SemaphoreType.DMA],
    **_MESH)


def _sc_hist(dst):
    h0, h1 = _hist_call()(dst)
    return h0.reshape(N, 1), h1.reshape(N, 1)


def _w_body(src_hbm, dst_hbm, al_hbm, ar_hbm, dis_hbm, w_hbm,
            src_v, dst_v, av, bv, sv, dv, wbuf, sem):
    c = lax.axis_index("c")
    s = lax.axis_index("s")
    wid = s * _NC + c
    nch = 390 + jnp.where(wid < _NCHUNK - 32 * 390, 1, 0)

    def chunk(i, _):
        base = (wid + i * 32) * _CH
        pltpu.sync_copy(src_hbm.at[pl.ds(base, _CH)], src_v)
        pltpu.sync_copy(dst_hbm.at[pl.ds(base, _CH)], dst_v)
        cp1 = pltpu.async_copy(al_hbm.at[src_v], av, sem)
        cp2 = pltpu.async_copy(dis_hbm.at[src_v], sv, sem)
        cp3 = pltpu.async_copy(ar_hbm.at[dst_v], bv, sem)
        cp4 = pltpu.async_copy(dis_hbm.at[dst_v], dv, sem)
        cp1.wait()
        cp2.wait()
        cp3.wait()
        cp4.wait()
        for j in range(8):
            sl = pl.ds(j * 16, 16)
            z = av[sl] + bv[sl]
            e = jnp.exp(z + z)
            t = 1.0 - 2.0 / (e + 1.0)
            wbuf[sl] = t * sv[sl] * dv[sl]
        pltpu.sync_copy(wbuf, w_hbm.at[pl.ds(base, _CH)])
        return 0

    lax.fori_loop(0, nch, chunk, 0)


_w_call = functools.partial(
    pl.kernel, _w_body,
    out_type=[jax.ShapeDtypeStruct((E,), _f32)],
    scratch_types=[pltpu.VMEM((_CH,), jnp.int32),
                   pltpu.VMEM((_CH,), jnp.int32),
                   pltpu.VMEM((_CH,), _f32),
                   pltpu.VMEM((_CH,), _f32),
                   pltpu.VMEM((_CH,), _f32),
                   pltpu.VMEM((_CH,), _f32),
                   pltpu.VMEM((_CH,), _f32),
                   pltpu.SemaphoreType.DMA],
    **_MESH)


def _sc_w(src, dst, al, ar, dis):
    return _w_call()(src, dst, al, ar, dis)[0]


_NH = N // 2
_TRASH = _NH


def _edge_half(src_hbm, dst_hbm, w_hbm, tab, out, s,
               src_v, dst_v, rt_v, w_v, rows_v, zbuf, acc, sem):
    for p in range(2):
        start_z = jnp.minimum(s * 3128, _NH + 8 - 3136)
        _zero_rows(zbuf, 3136, 1)
        pltpu.sync_copy(zbuf, acc.at[pl.ds(start_z, 3136)])
        plsc.subcore_barrier()

        nch = 781 + jnp.where(s < _NCHUNK - 16 * 781, 1, 0)
        lo = p * _NH

        def chunk(i, _):
            base = (s + i * _NS) * _CH
            pltpu.sync_copy(src_hbm.at[pl.ds(base, _CH)], src_v)
            pltpu.sync_copy(dst_hbm.at[pl.ds(base, _CH)], dst_v)
            pltpu.sync_copy(w_hbm.at[pl.ds(base, _CH)], w_v)
            for g in range(8):
                sl = pl.ds(g * 16, 16)
                r = dst_v[sl] - lo
                inb = jnp.logical_and(r >= 0, r < _NH)
                rt_v[sl] = jnp.where(inb, r, _TRASH)
            pltpu.async_copy(tab.at[src_v], rows_v, sem).wait()

            def scale(g, _):
                w16 = w_v[pl.ds(g * 16, 16)]
                for i in range(16):
                    wb = jnp.full((16,), w16[i], _f32)
                    j = g * 16 + i
                    rows_v[j, :] = rows_v[j, :] * wb
                return 0

            lax.fori_loop(0, _CH // 16, scale, 0)
            pltpu.sync_copy(rows_v, acc.at[rt_v], add=True)
            return 0

        lax.fori_loop(0, nch, chunk, 0)
        plsc.subcore_barrier()
        start_a = jnp.minimum((s * 3125) // 8 * 8, _NH - 3136)
        pltpu.sync_copy(acc.at[pl.ds(start_a, 3136)], zbuf)
        pltpu.sync_copy(zbuf, out.at[pl.ds(lo + start_a, 3136)])
        plsc.subcore_barrier()


def _edge_body(src_hbm, dst_hbm, w_hbm, xlo_hbm, xhi_hbm, outlo, outhi,
               src_v, dst_v, rt_v, w_v, rows_v, zbuf, acc, sem):
    c = lax.axis_index("c")
    s = lax.axis_index("s")

    @pl.when(c == 0)
    def _():
        _edge_half(src_hbm, dst_hbm, w_hbm, xlo_hbm, outlo, s,
                   src_v, dst_v, rt_v, w_v, rows_v, zbuf, acc, sem)

    @pl.when(c == 1)
    def _():
        _edge_half(src_hbm, dst_hbm, w_hbm, xhi_hbm, outhi, s,
                   src_v, dst_v, rt_v, w_v, rows_v, zbuf, acc, sem)


_edge_call = functools.partial(
    pl.kernel, _edge_body,
    out_type=[jax.ShapeDtypeStruct((N, H), _f32),
              jax.ShapeDtypeStruct((N, H), _f32)],
    scratch_types=[pltpu.VMEM((_CH,), jnp.int32),
                   pltpu.VMEM((_CH,), jnp.int32),
                   pltpu.VMEM((_CH,), jnp.int32),
                   pltpu.VMEM((_CH,), _f32),
                   pltpu.VMEM((_CH, H), _f32),
                   pltpu.VMEM((3136, H), _f32),
                   pltpu.VMEM_SHARED((_NH + 8, H), _f32),
                   pltpu.SemaphoreType.DMA],
    **_NOTILE, **_MESH)


def _sc_edge(src, dst, w, xlo, xhi):
    return _edge_call()(src, dst, w, xlo, xhi)


_SEGT = 160
_X3R = 32 * _SEGT


def _sload(ref, i):
    return ref[pl.ds(i, 16)][0]


def _seg_body(x2_hbm, trp_hbm, x3_hbm, trv, bnd, rbuf, outbuf, sem):
    c = lax.axis_index("c")
    s = lax.axis_index("s")
    wid = s * _NC + c
    s0 = wid * _SEGT
    s1 = jnp.minimum(s0 + _SEGT, B)
    nseg = s1 - s0
    pltpu.sync_copy(trp_hbm.at[pl.ds(s0, 176)], trv)
    iota = lax.iota(jnp.int32, 16)
    big = jnp.full((16,), 2 ** 30, jnp.int32)
    for g in range(10):
        v = trv[pl.ds(1 + g * 16, 16)]
        keep = (iota + g * 16) < (nseg - 1)
        bnd[pl.ds(g * 16, 16)] = jnp.where(keep, v, big)
    _zero_rows(outbuf, _SEGT, 2)

    r0 = trv[pl.ds(0, 16)][0]
    rend = _sload(trv, nseg)
    r0a = (r0 // 8) * 8
    nchunks = (rend - r0a + 63) // 64

    def chunk(ci, _):
        rb = r0a + ci * 64
        pltpu.sync_copy(x2_hbm.at[pl.ds(rb, 64)], rbuf)
        j0 = jnp.maximum(0, r0 - rb)
        m = jnp.minimum(64, rend - rb)

        def row(j, _):
            r = rb + j
            rv = jnp.full((16,), r, jnp.int32)
            c16 = jnp.zeros((16,), jnp.int32)
            for g in range(10):
                c16 = c16 + jnp.where(bnd[pl.ds(g * 16, 16)] <= rv, 1, 0)
            k = jnp.sum(c16)
            outbuf[k, pl.ds(0, 16)] = \
                outbuf[k, pl.ds(0, 16)] + rbuf[j, pl.ds(0, 16)]
            outbuf[k, pl.ds(16, 16)] = \
                outbuf[k, pl.ds(16, 16)] + rbuf[j, pl.ds(16, 16)]
            return 0

        lax.fori_loop(j0, m, row, 0)
        return 0

    lax.fori_loop(0, nchunks, chunk, 0)
    pltpu.sync_copy(outbuf, x3_hbm.at[pl.ds(s0, _SEGT)])


_seg_call = functools.partial(
    pl.kernel, _seg_body,
    out_type=[jax.ShapeDtypeStruct((_X3R, D1), _f32)],
    scratch_types=[pltpu.VMEM((176,), jnp.int32),
                   pltpu.VMEM((160,), jnp.int32),
                   pltpu.VMEM((64, D1), _f32),
                   pltpu.VMEM((_SEGT, D1), _f32),
                   pltpu.SemaphoreType.DMA],
    **_NOLAYOUT, **_MESH)


def _sc_seg(x2p, trp):
    return _seg_call()(x2p, trp)[0][:B]


def _rg_body(x3p_hbm, ri_hbm, out_hbm, idx_v, rows_v, idx_t, rows_t, sem):
    c = lax.axis_index("c")
    s = lax.axis_index("s")
    wid = s * _NC + c
    nfull = NUM_USERS // _CH
    ntail = NUM_USERS - nfull * _CH
    total = nfull + 1
    nch = total // 32 + jnp.where(wid < total - 32 * (total // 32), 1, 0)

    def chunk(i, _):
        cid = wid + i * 32

        @pl.when(cid < nfull)
        def _():
            base = cid * _CH
            pltpu.sync_copy(ri_hbm.at[pl.ds(base, _CH)], idx_v)
            pltpu.async_copy(x3p_hbm.at[idx_v], rows_v, sem).wait()
            pltpu.sync_copy(rows_v, out_hbm.at[pl.ds(base, _CH)])

        @pl.when(cid == nfull)
        def _():
            base = nfull * _CH
            pltpu.sync_copy(ri_hbm.at[pl.ds(base, ntail)], idx_t)
            pltpu.async_copy(x3p_hbm.at[idx_t], rows_t, sem).wait()
            pltpu.sync_copy(rows_t, out_hbm.at[pl.ds(base, ntail)])

        return 0

    lax.fori_loop(0, nch, chunk, 0)


_rg_call = functools.partial(
    pl.kernel, _rg_body,
    out_type=[jax.ShapeDtypeStruct((NUM_USERS, D1), _f32)],
    scratch_types=[pltpu.VMEM((_CH,), jnp.int32),
                   pltpu.VMEM((_CH, D1), _f32),
                   pltpu.VMEM((16,), jnp.int32),
                   pltpu.VMEM((16, D1), _f32),
                   pltpu.SemaphoreType.DMA],
    **_NOTILE, **_MESH)


def _sc_regather(x3p, re_index):
    return _rg_call()(x3p, re_index)[0]


def kernel(num_prop, cat_prop, tweet_range_list, edge_index, re_index,
           Wn, bn, Wc, bc, Wt, bt, att_l, att_r, Wf1, bf1, Wl, bl):
    attl_row = att_l.reshape(1, D1)
    attr_row = att_r.reshape(1, D1)
    src = edge_index[0]
    dst = edge_index[1]

    xlo, xhi, al, ar = _tc_feat(num_prop, cat_prop, Wn, bn.reshape(1, 16),
                                Wc, bc.reshape(1, 16), Wt, bt.reshape(1, 32),
                                attl_row, attr_row)
    h0, h1 = _sc_hist(dst)
    dis, sl1 = _tc_prep(h0, h1, al, ar)
    dis_f = dis.reshape(N)

    w1 = _sc_w(src, dst, al.reshape(N), ar.reshape(N), dis_f)
    aglo, aghi = _sc_edge(src, dst, w1, xlo, xhi)
    x1lo, x1hi, al2, ar2, sl2 = _tc_mix(aglo, aghi, xlo, xhi, sl1, dis,
                                        attl_row, attr_row)

    w2 = _sc_w(src, dst, al2.reshape(N), ar2.reshape(N), dis_f)
    ag2lo, ag2hi = _sc_edge(src, dst, w2, x1lo, x1hi)
    x2 = _tc_x2(ag2lo, ag2hi, x1lo, x1hi, xlo, xhi, sl2)

    x2p = jnp.concatenate([x2, jnp.zeros((64, D1), _f32)], axis=0)
    trp = jnp.concatenate(
        [tweet_range_list,
         jnp.full((_X3R + 168 - (B + 1),), N, tweet_range_list.dtype)])
    x3 = _sc_seg(x2p, trp)
    x3p = jnp.concatenate([x3, jnp.zeros((NUM_USERS - B, D1), _f32)], axis=0)
    x3g = _sc_regather(x3p, re_index)
    return _tc_mlp(x3g, Wf1, bf1.reshape(1, H), Wl, bl.reshape(1, 2))


# speedup vs baseline: 15.8002x; 1.7835x over previous
import functools

import jax
import jax.numpy as jnp
import numpy as np
from jax import lax
from jax.experimental import pallas as pl
from jax.experimental.pallas import tpu as pltpu
from jax.experimental.pallas import tpu_sc as plsc

N = 100000
E = 1600000
B = 5000
NUM_USERS = 10000
D1 = 32
H = D1 // 2
EPS = 0.1

_RB = 2000
_GRID = N // _RB

_f32 = jnp.float32


def _leaky(x):
    return jnp.where(x > 0, x, 0.01 * x)


def _rowspec(width):
    return pl.BlockSpec((_RB, width), lambda i: (i, 0))


def _fullspec(a, b):
    return pl.BlockSpec((a, b), lambda i: (0, 0))


def _feat_body(num_ref, cat_ref, wn_ref, bn_ref, wc_ref, bc_ref, wt_ref,
               bt_ref, attl_ref, attr_ref, xlo_ref, xhi_ref, al_ref, ar_ref):
    h1 = _leaky(jnp.dot(num_ref[...], wn_ref[...],
                        preferred_element_type=_f32) + bn_ref[...])
    h2 = _leaky(jnp.dot(cat_ref[...], wc_ref[...],
                        preferred_element_type=_f32) + bc_ref[...])
    x = jnp.concatenate([h1, h2], axis=1)
    x = _leaky(jnp.dot(x, wt_ref[...], preferred_element_type=_f32)
               + bt_ref[...])
    xlo_ref[...] = x[:, :H]
    xhi_ref[...] = x[:, H:]
    al_ref[...] = jnp.sum(x * attl_ref[...], axis=1, keepdims=True)
    ar_ref[...] = jnp.sum(x * attr_ref[...], axis=1, keepdims=True)


def _tc_feat(num, cat, Wn, bn, Wc, bc, Wt, bt, attl_row, attr_row):
    return pl.pallas_call(
        _feat_body,
        grid=(_GRID,),
        in_specs=[_rowspec(16), _rowspec(16), _fullspec(16, 16),
                  _fullspec(1, 16), _fullspec(16, 16), _fullspec(1, 16),
                  _fullspec(32, 32), _fullspec(1, 32), _fullspec(1, 32),
                  _fullspec(1, 32)],
        out_specs=[_rowspec(H), _rowspec(H), _rowspec(1), _rowspec(1)],
        out_shape=[jax.ShapeDtypeStruct((N, H), _f32),
                   jax.ShapeDtypeStruct((N, H), _f32),
                   jax.ShapeDtypeStruct((N, 1), _f32),
                   jax.ShapeDtypeStruct((N, 1), _f32)],
    )(num, cat, Wn, bn, Wc, bc, Wt, bt, attl_row, attr_row)


def _prep_body(h0_ref, h1_ref, al_ref, ar_ref, dis_ref, sl_ref):
    deg = h0_ref[...] + h1_ref[...] + 1.0
    dis = lax.rsqrt(deg)
    dis_ref[...] = dis
    sl_ref[...] = jnp.tanh(al_ref[...] + ar_ref[...]) / deg


def _tc_prep(h0, h1, al, ar):
    return pl.pallas_call(
        _prep_body,
        grid=(_GRID,),
        in_specs=[_rowspec(1)] * 4,
        out_specs=[_rowspec(1), _rowspec(1)],
        out_shape=[jax.ShapeDtypeStruct((N, 1), _f32),
                   jax.ShapeDtypeStruct((N, 1), _f32)],
    )(h0, h1, al, ar)


def _mix_body(aglo_ref, aghi_ref, xlo_ref, xhi_ref, sl_ref, dis_ref,
              attl_ref, attr_ref,
              ylo_ref, yhi_ref, al2_ref, ar2_ref, sl2_ref):
    s = sl_ref[...] + EPS
    ylo = aglo_ref[...] + xlo_ref[...] * s
    yhi = aghi_ref[...] + xhi_ref[...] * s
    ylo_ref[...] = ylo
    yhi_ref[...] = yhi
    y = jnp.concatenate([ylo, yhi], axis=1)
    al2 = jnp.sum(y * attl_ref[...], axis=1, keepdims=True)
    ar2 = jnp.sum(y * attr_ref[...], axis=1, keepdims=True)
    dis = dis_ref[...]
    al2_ref[...] = al2
    ar2_ref[...] = ar2
    sl2_ref[...] = jnp.tanh(al2 + ar2) * dis * dis


def _tc_mix(aglo, aghi, xlo, xhi, sl, dis, attl_row, attr_row):
    return pl.pallas_call(
        _mix_body,
        grid=(_GRID,),
        in_specs=[_rowspec(H), _rowspec(H), _rowspec(H), _rowspec(H),
                  _rowspec(1), _rowspec(1), _fullspec(1, 32), _fullspec(1, 32)],
        out_specs=[_rowspec(H), _rowspec(H), _rowspec(1), _rowspec(1),
                   _rowspec(1)],
        out_shape=[jax.ShapeDtypeStruct((N, H), _f32),
                   jax.ShapeDtypeStruct((N, H), _f32),
                   jax.ShapeDtypeStruct((N, 1), _f32),
                   jax.ShapeDtypeStruct((N, 1), _f32),
                   jax.ShapeDtypeStruct((N, 1), _f32)],
    )(aglo, aghi, xlo, xhi, sl, dis, attl_row, attr_row)


def _x2_body(aglo_ref, aghi_ref, x1lo_ref, x1hi_ref, xlo_ref, xhi_ref,
             sl2_ref, x2_ref):
    s = sl2_ref[...]
    lo = aglo_ref[...] + x1lo_ref[...] * s + EPS * xlo_ref[...]
    hi = aghi_ref[...] + x1hi_ref[...] * s + EPS * xhi_ref[...]
    x2 = jnp.concatenate([lo, hi], axis=1)
    x2_ref[...] = jnp.sqrt(x2 * x2 + 1e-8)


def _tc_x2(aglo, aghi, x1lo, x1hi, xlo, xhi, sl2):
    return pl.pallas_call(
        _x2_body,
        grid=(_GRID,),
        in_specs=[_rowspec(H)] * 4 + [_rowspec(H), _rowspec(H), _rowspec(1)],
        out_specs=[pl.BlockSpec((_RB, D1), lambda i: (i, 0))],
        out_shape=[jax.ShapeDtypeStruct((N, D1), _f32)],
    )(aglo, aghi, x1lo, x1hi, xlo, xhi, sl2)[0]


def _mlp_body(x_ref, wf_ref, bf_ref, wl_ref, bl_ref, o_ref):
    h = _leaky(jnp.dot(x_ref[...], wf_ref[...],
                       preferred_element_type=_f32) + bf_ref[...])
    o_ref[...] = jnp.dot(h, wl_ref[...], preferred_element_type=_f32) \
        + bl_ref[...]


def _tc_mlp(x3g, Wf1, bf1_row, Wl, bl_row):
    return pl.pallas_call(
        _mlp_body,
        grid=(1,),
        in_specs=[pl.BlockSpec((NUM_USERS, D1), lambda i: (0, 0)),
                  _fullspec(D1, H), _fullspec(1, H), _fullspec(H, 2),
                  _fullspec(1, 2)],
        out_specs=[pl.BlockSpec((NUM_USERS, 2), lambda i: (0, 0))],
        out_shape=[jax.ShapeDtypeStruct((NUM_USERS, 2), _f32)],
    )(x3g, Wf1, bf1_row, Wl, bl_row)[0]


_NC, _NS, _L = 2, 16, 16
_CH = 128
_NCHUNK = E // _CH
_NPT = N // _NS
_MESH = dict(mesh=plsc.VectorSubcoreMesh(core_axis_name="c",
                                         subcore_axis_name="s"))
_NOTILE = dict(compiler_params=pltpu.CompilerParams(
    use_tc_tiling_on_sc=False))
_NOLAYOUT = dict(compiler_params=pltpu.CompilerParams(
    needs_layout_passes=False))


def _zero_rows(buf, nrows, width16):
    z = jnp.zeros((16,), _f32)

    def body(j, _):
        for col in range(width16):
            buf[j, pl.ds(col * 16, 16)] = z
        return 0

    lax.fori_loop(0, nrows, body, 0)


def _hist_body(dst_hbm, h0_hbm, h1_hbm, dst_v, ones_v, zb_v, acc, sem):
    c = lax.axis_index("c")
    s = lax.axis_index("s")
    def fill16(j, _):
        zb_v[pl.ds(j * 16, 16)] = jnp.zeros((16,), _f32)
        return 0
    lax.fori_loop(0, 391, fill16, 0)
    for j in range(8):
        ones_v[pl.ds(j * 16, 16)] = jnp.ones((16,), _f32)
    start_a = jnp.minimum((s * _NPT) // 8 * 8, N - 6256)
    pltpu.sync_copy(zb_v, acc.at[pl.ds(start_a, 6256)])
    plsc.subcore_barrier()

    half = _NCHUNK // 2
    nch = 390 + jnp.where(s < half - 16 * 390, 1, 0)

    def chunk(i, _):
        base = (c * half + s + i * _NS) * _CH
        pltpu.sync_copy(dst_hbm.at[pl.ds(base, _CH)], dst_v)
        pltpu.sync_copy(ones_v, acc.at[dst_v], add=True)
        return 0

    lax.fori_loop(0, nch, chunk, 0)
    plsc.subcore_barrier()

    pltpu.sync_copy(acc.at[pl.ds(start_a, 6256)], zb_v)

    @pl.when(c == 0)
    def _():
        pltpu.sync_copy(zb_v, h0_hbm.at[pl.ds(start_a, 6256)])

    @pl.when(c == 1)
    def _():
        pltpu.sync_copy(zb_v, h1_hbm.at[pl.ds(start_a, 6256)])


_hist_call = functools.partial(
    pl.kernel, _hist_body,
    out_type=[jax.ShapeDtypeStruct((N,), _f32),
              jax.ShapeDtypeStruct((N,), _f32)],
    scratch_types=[pltpu.VMEM((_CH,), jnp.int32),
                   pltpu.VMEM((_CH,), _f32),
                   pltpu.VMEM((6256,), _f32),
                   pltpu.VMEM_SHARED((N,), _f32),
                   pltpu.SemaphoreType.DMA],
    **_MESH)


def _sc_hist(dst):
    h0, h1 = _hist_call()(dst)
    return h0.reshape(N, 1), h1.reshape(N, 1)


_SCH = 4
_NSUP = _NCHUNK // _SCH


def _w_body(src_hbm, dst_hbm, al_hbm, ar_hbm, dis_hbm, w_hbm,
            src_v, dst_v, av, bv, sv, dv, wbuf, sem):
    c = lax.axis_index("c")
    s = lax.axis_index("s")
    wid = s * _NC + c
    nch = 97 + jnp.where(wid < _NSUP - 32 * 97, 1, 0)

    def chunk(i, _):
        cb = (wid + i * 32) * _SCH
        pltpu.sync_copy(src_hbm.at[pl.ds(cb, _SCH)], src_v)
        pltpu.sync_copy(dst_hbm.at[pl.ds(cb, _SCH)], dst_v)
        cps = []
        for j in range(_SCH):
            cps.append(pltpu.async_copy(al_hbm.at[src_v.at[j]],
                                        av.at[j], sem))
            cps.append(pltpu.async_copy(dis_hbm.at[src_v.at[j]],
                                        sv.at[j], sem))
            cps.append(pltpu.async_copy(ar_hbm.at[dst_v.at[j]],
                                        bv.at[j], sem))
            cps.append(pltpu.async_copy(dis_hbm.at[dst_v.at[j]],
                                        dv.at[j], sem))
        for cp in cps:
            cp.wait()
        for j in range(_SCH):
            for g in range(8):
                sl = pl.ds(g * 16, 16)
                z = av[j, sl] + bv[j, sl]
                e = jnp.exp(z + z)
                t = 1.0 - 2.0 / (e + 1.0)
                wbuf[j, sl] = t * sv[j, sl] * dv[j, sl]
        pltpu.sync_copy(wbuf, w_hbm.at[pl.ds(cb, _SCH)])
        return 0

    lax.fori_loop(0, nch, chunk, 0)


_w_call = functools.partial(
    pl.kernel, _w_body,
    out_type=[jax.ShapeDtypeStruct((_NCHUNK, _CH), _f32)],
    scratch_types=[pltpu.VMEM((_SCH, _CH), jnp.int32),
                   pltpu.VMEM((_SCH, _CH), jnp.int32),
                   pltpu.VMEM((_SCH, _CH), _f32),
                   pltpu.VMEM((_SCH, _CH), _f32),
                   pltpu.VMEM((_SCH, _CH), _f32),
                   pltpu.VMEM((_SCH, _CH), _f32),
                   pltpu.VMEM((_SCH, _CH), _f32),
                   pltpu.SemaphoreType.DMA],
    **_MESH)


def _sc_w(src2, dst2, al, ar, dis):
    return _w_call()(src2, dst2, al, ar, dis)[0]


_NH = N // 2
_TRASH = _NH


def _edge_half(src_hbm, dst_hbm, w_hbm, tab, out, s,
               src_v, rt_v, w_v, rows_v, zbuf, acc, sem, sem2):
    for p in range(2):
        start_z = jnp.minimum(s * 3128, _NH + 8 - 3136)
        _zero_rows(zbuf, 3136, 1)
        pltpu.sync_copy(zbuf, acc.at[pl.ds(start_z, 3136)])
        plsc.subcore_barrier()

        nch = 195 + jnp.where(s < _NSUP - 16 * 195, 1, 0)
        lo = p * _NH

        def chunk(i, _):
            cb = (s + i * _NS) * _SCH
            pltpu.sync_copy(src_hbm.at[pl.ds(cb, _SCH)], src_v)
            pltpu.sync_copy(dst_hbm.at[pl.ds(cb, _SCH)], rt_v)
            pltpu.sync_copy(w_hbm.at[pl.ds(cb, _SCH)], w_v)
            cps = [pltpu.async_copy(tab.at[src_v.at[j]],
                                    rows_v.at[pl.ds(j * _CH, _CH)], sem)
                   for j in range(_SCH)]
            for j in range(_SCH):
                for g in range(8):
                    sl = pl.ds(g * 16, 16)
                    r = rt_v[j, sl] - lo
                    inb = jnp.logical_and(r >= 0, r < _NH)
                    rt_v[j, sl] = jnp.where(inb, r, _TRASH)
            for cp in cps:
                cp.wait()

            def scale(g, _):
                j = g // 8
                w16 = w_v[j, pl.ds((g % 8) * 16, 16)]
                for i in range(16):
                    wb = jnp.full((16,), w16[i], _f32)
                    rr = g * 16 + i
                    rows_v[rr, :] = rows_v[rr, :] * wb
                return 0

            lax.fori_loop(0, (_SCH * _CH) // 16, scale, 0)
            scps = [pltpu.async_copy(rows_v.at[pl.ds(j * _CH, _CH)],
                                     acc.at[rt_v.at[j]], sem2, add=True)
                    for j in range(_SCH)]
            for cp in scps:
                cp.wait()
            return 0

        lax.fori_loop(0, nch, chunk, 0)
        plsc.subcore_barrier()
        start_a = jnp.minimum((s * 3125) // 8 * 8, _NH - 3136)
        pltpu.sync_copy(acc.at[pl.ds(start_a, 3136)], zbuf)
        pltpu.sync_copy(zbuf, out.at[pl.ds(lo + start_a, 3136)])
        plsc.subcore_barrier()


def _edge_body(src_hbm, dst_hbm, w_hbm, xlo_hbm, xhi_hbm, outlo, outhi,
               src_v, rt_v, w_v, rows_v, zbuf, acc, sem, sem2):
    c = lax.axis_index("c")
    s = lax.axis_index("s")

    @pl.when(c == 0)
    def _():
        _edge_half(src_hbm, dst_hbm, w_hbm, xlo_hbm, outlo, s,
                   src_v, rt_v, w_v, rows_v, zbuf, acc, sem, sem2)

    @pl.when(c == 1)
    def _():
        _edge_half(src_hbm, dst_hbm, w_hbm, xhi_hbm, outhi, s,
                   src_v, rt_v, w_v, rows_v, zbuf, acc, sem, sem2)


_edge_call = functools.partial(
    pl.kernel, _edge_body,
    out_type=[jax.ShapeDtypeStruct((N, H), _f32),
              jax.ShapeDtypeStruct((N, H), _f32)],
    scratch_types=[pltpu.VMEM((_SCH, _CH), jnp.int32),
                   pltpu.VMEM((_SCH, _CH), jnp.int32),
                   pltpu.VMEM((_SCH, _CH), _f32),
                   pltpu.VMEM((_SCH * _CH, H), _f32),
                   pltpu.VMEM((3136, H), _f32),
                   pltpu.VMEM_SHARED((_NH + 8, H), _f32),
                   pltpu.SemaphoreType.DMA,
                   pltpu.SemaphoreType.DMA],
    **_NOTILE, **_MESH)


def _sc_edge(src2, dst2, w2, xlo, xhi):
    return _edge_call()(src2, dst2, w2, xlo, xhi)


_SEGT = 160
_X3R = 32 * _SEGT


def _sload(ref, i):
    return ref[pl.ds(i, 16)][0]


def _seg_body(x2_hbm, trp_hbm, x3_hbm, trv, bnd, rbuf, outbuf, sem):
    c = lax.axis_index("c")
    s = lax.axis_index("s")
    wid = s * _NC + c
    s0 = wid * _SEGT
    s1 = jnp.minimum(s0 + _SEGT, B)
    nseg = s1 - s0
    pltpu.sync_copy(trp_hbm.at[pl.ds(s0, 176)], trv)
    iota = lax.iota(jnp.int32, 16)
    big = jnp.full((16,), 2 ** 30, jnp.int32)
    for g in range(10):
        v = trv[pl.ds(1 + g * 16, 16)]
        keep = (iota + g * 16) < (nseg - 1)
        bnd[pl.ds(g * 16, 16)] = jnp.where(keep, v, big)
    _zero_rows(outbuf, _SEGT, 2)

    r0 = trv[pl.ds(0, 16)][0]
    rend = _sload(trv, nseg)
    r0a = (r0 // 8) * 8
    nchunks = (rend - r0a + 63) // 64

    def chunk(ci, _):
        rb = r0a + ci * 64
        pltpu.sync_copy(x2_hbm.at[pl.ds(rb, 64)], rbuf)
        j0 = jnp.maximum(0, r0 - rb)
        m = jnp.minimum(64, rend - rb)

        def row(j, _):
            r = rb + j
            rv = jnp.full((16,), r, jnp.int32)
            c16 = jnp.zeros((16,), jnp.int32)
            for g in range(10):
                c16 = c16 + jnp.where(bnd[pl.ds(g * 16, 16)] <= rv, 1, 0)
            k = jnp.sum(c16)
            outbuf[k, pl.ds(0, 16)] = \
                outbuf[k, pl.ds(0, 16)] + rbuf[j, pl.ds(0, 16)]
            outbuf[k, pl.ds(16, 16)] = \
                outbuf[k, pl.ds(16, 16)] + rbuf[j, pl.ds(16, 16)]
            return 0

        lax.fori_loop(j0, m, row, 0)
        return 0

    lax.fori_loop(0, nchunks, chunk, 0)
    pltpu.sync_copy(outbuf, x3_hbm.at[pl.ds(s0, _SEGT)])


_seg_call = functools.partial(
    pl.kernel, _seg_body,
    out_type=[jax.ShapeDtypeStruct((_X3R, D1), _f32)],
    scratch_types=[pltpu.VMEM((176,), jnp.int32),
                   pltpu.VMEM((160,), jnp.int32),
                   pltpu.VMEM((64, D1), _f32),
                   pltpu.VMEM((_SEGT, D1), _f32),
                   pltpu.SemaphoreType.DMA],
    **_NOLAYOUT, **_MESH)


def _sc_seg(x2p, trp):
    return _seg_call()(x2p, trp)[0][:B]


def _rg_body(x3p_hbm, ri_hbm, out_hbm, idx_v, rows_v, idx_t, rows_t, sem):
    c = lax.axis_index("c")
    s = lax.axis_index("s")
    wid = s * _NC + c
    nfull = NUM_USERS // _CH
    ntail = NUM_USERS - nfull * _CH
    total = nfull + 1
    nch = total // 32 + jnp.where(wid < total - 32 * (total // 32), 1, 0)

    def chunk(i, _):
        cid = wid + i * 32

        @pl.when(cid < nfull)
        def _():
            base = cid * _CH
            pltpu.sync_copy(ri_hbm.at[pl.ds(base, _CH)], idx_v)
            pltpu.async_copy(x3p_hbm.at[idx_v], rows_v, sem).wait()
            pltpu.sync_copy(rows_v, out_hbm.at[pl.ds(base, _CH)])

        @pl.when(cid == nfull)
        def _():
            base = nfull * _CH
            pltpu.sync_copy(ri_hbm.at[pl.ds(base, ntail)], idx_t)
            pltpu.async_copy(x3p_hbm.at[idx_t], rows_t, sem).wait()
            pltpu.sync_copy(rows_t, out_hbm.at[pl.ds(base, ntail)])

        return 0

    lax.fori_loop(0, nch, chunk, 0)


_rg_call = functools.partial(
    pl.kernel, _rg_body,
    out_type=[jax.ShapeDtypeStruct((NUM_USERS, D1), _f32)],
    scratch_types=[pltpu.VMEM((_CH,), jnp.int32),
                   pltpu.VMEM((_CH, D1), _f32),
                   pltpu.VMEM((16,), jnp.int32),
                   pltpu.VMEM((16, D1), _f32),
                   pltpu.SemaphoreType.DMA],
    **_NOTILE, **_MESH)


def _sc_regather(x3p, re_index):
    return _rg_call()(x3p, re_index)[0]


def kernel(num_prop, cat_prop, tweet_range_list, edge_index, re_index,
           Wn, bn, Wc, bc, Wt, bt, att_l, att_r, Wf1, bf1, Wl, bl):
    attl_row = att_l.reshape(1, D1)
    attr_row = att_r.reshape(1, D1)
    src = edge_index[0]
    dst = edge_index[1]

    xlo, xhi, al, ar = _tc_feat(num_prop, cat_prop, Wn, bn.reshape(1, 16),
                                Wc, bc.reshape(1, 16), Wt, bt.reshape(1, 32),
                                attl_row, attr_row)
    src2 = src.reshape(_NCHUNK, _CH)
    dst2 = dst.reshape(_NCHUNK, _CH)

    h0, h1 = _sc_hist(dst)
    dis, sl1 = _tc_prep(h0, h1, al, ar)
    dis_f = dis.reshape(N)

    w1 = _sc_w(src2, dst2, al.reshape(N), ar.reshape(N), dis_f)
    aglo, aghi = _sc_edge(src2, dst2, w1, xlo, xhi)
    x1lo, x1hi, al2, ar2, sl2 = _tc_mix(aglo, aghi, xlo, xhi, sl1, dis,
                                        attl_row, attr_row)

    w2 = _sc_w(src2, dst2, al2.reshape(N), ar2.reshape(N), dis_f)
    ag2lo, ag2hi = _sc_edge(src2, dst2, w2, x1lo, x1hi)
    x2 = _tc_x2(ag2lo, ag2hi, x1lo, x1hi, xlo, xhi, sl2)

    x2p = jnp.concatenate([x2, jnp.zeros((64, D1), _f32)], axis=0)
    trp = jnp.concatenate(
        [tweet_range_list,
         jnp.full((_X3R + 168 - (B + 1),), N, tweet_range_list.dtype)])
    x3 = _sc_seg(x2p, trp)
    x3p = jnp.concatenate([x3, jnp.zeros((NUM_USERS - B, D1), _f32)], axis=0)
    x3g = _sc_regather(x3p, re_index)
    return _tc_mlp(x3g, Wf1, bf1.reshape(1, H), Wl, bl.reshape(1, 2))
